# baseline (speedup 1.0000x reference)
import functools

import jax
import jax.numpy as jnp
from jax import lax
from jax.experimental import pallas as pl
from jax.experimental.pallas import tpu as pltpu
from jax.experimental.pallas import tpu_sc as plsc

NUM = 20
TEMP = 0.07
B, C, H, W = 16, 256, 65, 65
HW = H * W
IN_HW = 513
KPAD = 32

NC, NS = 2, 16
NW = NC * NS
ROWS = B * H
ROWS_PER_W = 33
ROW_PAD = NW * ROWS_PER_W
WPAD = 80
ROW_BUF = 528


def _sc_gather_body(prop_hbm, out_hbm, row_v, out_v, sem):
    wid = lax.axis_index("s") * NC + lax.axis_index("c")
    base = wid * ROWS_PER_W

    descs = []
    shifts = []
    for t in range(ROWS_PER_W):
        r = jnp.minimum(base + t, ROWS - 1)
        b = r // H
        i = r % H
        ih = (i * IN_HW) // H
        off = b * (IN_HW * IN_HW) + ih * IN_HW
        start = (off // 8) * 8
        shifts.append(off - start)
        descs.append(pltpu.async_copy(
            prop_hbm.at[pl.ds(start, ROW_BUF)],
            row_v.at[pl.ds(t * ROW_BUF, ROW_BUF)], sem))
    for d in descs:
        d.wait()

    for t in range(ROWS_PER_W):
        sh = shifts[t]
        for k in range(5):
            jv = lax.iota(jnp.int32, 16) + (16 * k)
            iwv = (jv * IN_HW) // H
            idx = jnp.minimum(iwv + sh, ROW_BUF - 1) + (t * ROW_BUF)
            vals = plsc.load_gather(row_v, [idx])
            out_v[pl.ds(t * WPAD + 16 * k, 16)] = vals

    pltpu.sync_copy(out_v, out_hbm.at[pl.ds(base * WPAD, ROWS_PER_W * WPAD)])


@jax.jit
def _sc_downsample(prop_flat):
    fn = functools.partial(
        pl.kernel,
        mesh=plsc.VectorSubcoreMesh(core_axis_name="c", subcore_axis_name="s"),
        compiler_params=pltpu.CompilerParams(needs_layout_passes=False),
        out_type=jax.ShapeDtypeStruct((ROW_PAD * WPAD,), jnp.int32),
        scratch_types=[
            pltpu.VMEM((ROWS_PER_W * ROW_BUF,), jnp.int32),
            pltpu.VMEM((ROWS_PER_W * WPAD,), jnp.int32),
            pltpu.SemaphoreType.DMA,
        ],
    )(_sc_gather_body)
    return fn(prop_flat)


def _tc_body(feats_ref, prev_ref, lab_ref, out_ref, acc_anc, acc_con, cnt_ref):
    bidx = pl.program_id(0)

    @pl.when(bidx == 0)
    def _init():
        acc_anc[...] = jnp.zeros_like(acc_anc)
        acc_con[...] = jnp.zeros_like(acc_con)
        cnt_ref[...] = jnp.zeros_like(cnt_ref)

    lab = lab_ref[0]
    kiota = lax.broadcasted_iota(jnp.int32, (HW, KPAD), 1)
    onehot = (lab == kiota).astype(jnp.float32)

    x = feats_ref[0].reshape(C, HW)
    inv = 1.0 / jnp.maximum(
        jnp.sqrt(jnp.sum(x * x, axis=0, keepdims=True)), 1e-12)
    acc_anc[...] += jnp.dot(x * inv, onehot,
                            preferred_element_type=jnp.float32)

    xp = prev_ref[0].reshape(C, HW)
    invp = 1.0 / jnp.maximum(
        jnp.sqrt(jnp.sum(xp * xp, axis=0, keepdims=True)), 1e-12)
    acc_con[...] += jnp.dot(xp * invp, onehot,
                            preferred_element_type=jnp.float32)

    cnt_ref[...] += jnp.sum(onehot, axis=0, keepdims=True)

    @pl.when(bidx == B - 1)
    def _finish():
        denom = jnp.maximum(cnt_ref[...], 1.0)
        kvalid = lax.broadcasted_iota(jnp.int32, (1, KPAD), 1) < NUM
        ancT = jnp.where(kvalid, acc_anc[...] / denom, 0.0)
        conT = jnp.where(kvalid, acc_con[...] / denom, 0.0)
        contrastT = jnp.concatenate([ancT, conT], axis=1)
        anc = jnp.transpose(ancT)
        adc = jnp.dot(anc, contrastT,
                      preferred_element_type=jnp.float32) / TEMP

        ii = lax.broadcasted_iota(jnp.int32, (KPAD, 2 * KPAD), 0)
        jj = lax.broadcasted_iota(jnp.int32, (KPAD, 2 * KPAD), 1)
        jlab = jnp.where(jj < KPAD, jj, jj - KPAD)
        ivalid = ii < NUM
        jvalid = jlab < NUM
        vvalid = ivalid & jvalid
        r_mask = (vvalid & (ii == jlab)).astype(jnp.float32)
        eye = (vvalid & (jj < KPAD) & (ii == jj)).astype(jnp.float32)
        pos_mask = r_mask - eye
        neg_mask = jnp.where(vvalid, 1.0 - r_mask, 0.0)

        neg_contrast = jnp.sum(jnp.exp(adc) * neg_mask, axis=1, keepdims=True)
        logits_max = jnp.max(jnp.where(jvalid, adc, -1e30), axis=1,
                             keepdims=True)
        adc2 = adc - logits_max
        pos_contrast = (adc2 * pos_mask
                        - jnp.log(jnp.exp(adc2) + neg_contrast) * pos_mask)
        npos = jnp.sum(pos_mask, axis=1, keepdims=True)
        per = jnp.sum(pos_contrast, axis=1, keepdims=True)
        has = npos > 0
        loss_vec = jnp.where(has, -per / jnp.maximum(npos, 1.0), 0.0)
        num = jnp.sum(loss_vec, axis=0, keepdims=True)
        den = jnp.sum(has.astype(jnp.float32), axis=0, keepdims=True)
        out_ref[...] = num / jnp.maximum(den, 1.0)


@jax.jit
def _tc_main(feats, feats_prev, labels):
    return pl.pallas_call(
        _tc_body,
        grid=(B,),
        in_specs=[
            pl.BlockSpec((1, C, H, W), lambda b: (b, 0, 0, 0)),
            pl.BlockSpec((1, C, H, W), lambda b: (b, 0, 0, 0)),
            pl.BlockSpec((1, HW, 1), lambda b: (b, 0, 0)),
        ],
        out_specs=pl.BlockSpec((1, 1), lambda b: (0, 0)),
        out_shape=jax.ShapeDtypeStruct((1, 1), jnp.float32),
        scratch_shapes=[
            pltpu.VMEM((C, KPAD), jnp.float32),
            pltpu.VMEM((C, KPAD), jnp.float32),
            pltpu.VMEM((1, KPAD), jnp.float32),
        ],
    )(feats, feats_prev, labels)


def kernel(pre_logits, pre_logits_prev, proposal):
    sc_out = _sc_downsample(proposal.reshape(-1)).reshape(ROW_PAD, WPAD)
    labels = sc_out[:ROWS, :W].reshape(B, HW, 1)
    return _tc_main(pre_logits, pre_logits_prev, labels)[0, 0]


# speedup vs baseline: 1.9032x; 1.9032x over previous
import functools

import jax
import jax.numpy as jnp
from jax import lax
from jax.experimental import pallas as pl
from jax.experimental.pallas import tpu as pltpu
from jax.experimental.pallas import tpu_sc as plsc

NUM = 20
TEMP = 0.07
B, C, H, W = 16, 256, 65, 65
HW = H * W
NPIX = B * HW
IN_HW = 513
KPAD = 32

NC, NS = 2, 16
NW = NC * NS
GCHUNK = 128
NGATHER = 17
WCHUNK = GCHUNK * NGATHER
GOUT = NW * WCHUNK


def _sc_gather_body(prop_hbm, out_hbm, idx_v, val_v, sem):
    wid = lax.axis_index("s") * NC + lax.axis_index("c")
    base = wid * WCHUNK

    bvec = lax.iota(jnp.int32, 16) * (IN_HW * IN_HW)
    base16 = wid * (WCHUNK // 16)

    def _fill(t, carry):
        ij = jnp.minimum(base16 + t, HW - 1)
        i = ij // H
        j = ij - i * H
        ih = (i * IN_HW) // H
        iw = (j * IN_HW) // H
        idx_v[pl.ds(16 * t, 16)] = bvec + (ih * IN_HW + iw)
        return carry

    lax.fori_loop(0, WCHUNK // 16, _fill, 0)

    descs = []
    for g in range(NGATHER):
        descs.append(pltpu.async_copy(
            prop_hbm.at[idx_v.at[pl.ds(g * GCHUNK, GCHUNK)]],
            val_v.at[pl.ds(g * GCHUNK, GCHUNK)], sem))
    for d in descs:
        d.wait()

    pltpu.sync_copy(val_v, out_hbm.at[pl.ds(base, WCHUNK)])


@jax.jit
def _sc_downsample(prop_flat):
    fn = functools.partial(
        pl.kernel,
        mesh=plsc.VectorSubcoreMesh(core_axis_name="c", subcore_axis_name="s"),
        compiler_params=pltpu.CompilerParams(needs_layout_passes=False),
        out_type=jax.ShapeDtypeStruct((GOUT,), jnp.int32),
        scratch_types=[
            pltpu.VMEM((WCHUNK,), jnp.int32),
            pltpu.VMEM((WCHUNK,), jnp.int32),
            pltpu.SemaphoreType.DMA,
        ],
    )(_sc_gather_body)
    return fn(prop_flat)


HB = 5
NSTEP = H // HB
PB = HB * W * B


def _tc_body(feats_ref, prev_ref, lab_ref, out_ref, acc_anc, acc_con, cnt_ref):
    step = pl.program_id(0)

    @pl.when(step == 0)
    def _init():
        acc_anc[...] = jnp.zeros_like(acc_anc)
        acc_con[...] = jnp.zeros_like(acc_con)
        cnt_ref[...] = jnp.zeros_like(cnt_ref)

    lab = lab_ref[0]
    kiota = lax.broadcasted_iota(jnp.int32, (KPAD, PB), 0)
    oh = (lab == kiota).astype(jnp.float32)
    ones_c = jnp.ones((1, C), jnp.float32)
    ones_p = jnp.ones((PB, 1), jnp.float32)

    x = feats_ref[...].reshape(PB, C)
    xx = x * x
    sumsq = lax.dot_general(ones_c, xx, (((1,), (1,)), ((), ())),
                            preferred_element_type=jnp.float32)
    inv = 1.0 / jnp.maximum(jnp.sqrt(sumsq), 1e-12)
    acc_anc[...] += lax.dot_general(oh * inv, x, (((1,), (0,)), ((), ())),
                                    preferred_element_type=jnp.float32)

    xp = prev_ref[...].reshape(PB, C)
    xxp = xp * xp
    sumsqp = lax.dot_general(ones_c, xxp, (((1,), (1,)), ((), ())),
                             preferred_element_type=jnp.float32)
    invp = 1.0 / jnp.maximum(jnp.sqrt(sumsqp), 1e-12)
    acc_con[...] += lax.dot_general(oh * invp, xp, (((1,), (0,)), ((), ())),
                                    preferred_element_type=jnp.float32)

    cnt_ref[...] += jnp.dot(oh, ones_p,
                            preferred_element_type=jnp.float32)

    @pl.when(step == NSTEP - 1)
    def _finish():
        denom = jnp.maximum(cnt_ref[...], 1.0)
        kvalid = lax.broadcasted_iota(jnp.int32, (KPAD, 1), 0) < NUM
        anc = jnp.where(kvalid, acc_anc[...] / denom, 0.0)
        con = jnp.where(kvalid, acc_con[...] / denom, 0.0)
        contrast = jnp.concatenate([anc, con], axis=0)
        adc = lax.dot_general(anc, contrast, (((1,), (1,)), ((), ())),
                              preferred_element_type=jnp.float32) / TEMP

        ii = lax.broadcasted_iota(jnp.int32, (KPAD, 2 * KPAD), 0)
        jj = lax.broadcasted_iota(jnp.int32, (KPAD, 2 * KPAD), 1)
        jlab = jnp.where(jj < KPAD, jj, jj - KPAD)
        ivalid = ii < NUM
        jvalid = jlab < NUM
        vvalid = ivalid & jvalid
        r_mask = (vvalid & (ii == jlab)).astype(jnp.float32)
        eye = (vvalid & (jj < KPAD) & (ii == jj)).astype(jnp.float32)
        pos_mask = r_mask - eye
        neg_mask = jnp.where(vvalid, 1.0 - r_mask, 0.0)

        neg_contrast = jnp.sum(jnp.exp(adc) * neg_mask, axis=1, keepdims=True)
        logits_max = jnp.max(jnp.where(jvalid, adc, -1e30), axis=1,
                             keepdims=True)
        adc2 = adc - logits_max
        pos_contrast = (adc2 * pos_mask
                        - jnp.log(jnp.exp(adc2) + neg_contrast) * pos_mask)
        npos = jnp.sum(pos_mask, axis=1, keepdims=True)
        per = jnp.sum(pos_contrast, axis=1, keepdims=True)
        has = npos > 0
        loss_vec = jnp.where(has, -per / jnp.maximum(npos, 1.0), 0.0)
        num = jnp.sum(loss_vec, axis=0, keepdims=True)
        den = jnp.sum(has.astype(jnp.float32), axis=0, keepdims=True)
        out_ref[...] = num / jnp.maximum(den, 1.0)


@jax.jit
def _tc_main(featsT, featsT_prev, labels):
    return pl.pallas_call(
        _tc_body,
        grid=(NSTEP,),
        in_specs=[
            pl.BlockSpec((HB, W, B, C), lambda s: (s, 0, 0, 0)),
            pl.BlockSpec((HB, W, B, C), lambda s: (s, 0, 0, 0)),
            pl.BlockSpec((1, 1, PB), lambda s: (s, 0, 0)),
        ],
        out_specs=pl.BlockSpec((1, 1), lambda s: (0, 0)),
        out_shape=jax.ShapeDtypeStruct((1, 1), jnp.float32),
        scratch_shapes=[
            pltpu.VMEM((KPAD, C), jnp.float32),
            pltpu.VMEM((KPAD, C), jnp.float32),
            pltpu.VMEM((KPAD, 1), jnp.float32),
        ],
    )(featsT, featsT_prev, labels)


def kernel(pre_logits, pre_logits_prev, proposal):
    labels = _sc_downsample(proposal.reshape(-1))[:NPIX].reshape(NSTEP, 1, PB)
    featsT = jnp.transpose(pre_logits, (2, 3, 0, 1))
    featsT_prev = jnp.transpose(pre_logits_prev, (2, 3, 0, 1))
    return _tc_main(featsT, featsT_prev, labels)[0, 0]


# speedup vs baseline: 7.2185x; 3.7929x over previous
import functools

import jax
import jax.numpy as jnp
from jax import lax
from jax.experimental import pallas as pl
from jax.experimental.pallas import tpu as pltpu
from jax.experimental.pallas import tpu_sc as plsc

NUM = 20
TEMP = 0.07
B, C, H, W = 16, 256, 65, 65
HW = H * W
NPIX = B * HW
IN_HW = 513
KPAD = 32

NC, NS = 2, 16
NW = NC * NS
GCHUNK = 128
NGATHER = 17
WCHUNK = GCHUNK * NGATHER
GOUT = NW * WCHUNK


def _sc_gather_body(prop_hbm, out_hbm, val_v, sem):
    wid = lax.axis_index("s") * NC + lax.axis_index("c")
    base16 = wid * (WCHUNK // 16)
    i0 = base16 // H

    def inner(buf):
        descs = []
        for rr in range(4):
            i = jnp.minimum(i0 + rr, H - 1)
            r = (i * IN_HW) // H
            for tb in range(2):
                for tc in range(4):
                    k = (rr * 2 + tb) * 4 + tc
                    descs.append(pltpu.async_copy(
                        prop_hbm.at[r, pl.ds(tb * 8, 8), pl.ds(tc * 128, 128)],
                        buf.at[pl.ds(k * 8, 8), :], sem))
        for d in descs:
            d.wait()

        tbv32 = (lax.iota(jnp.int32, 16) >> 3) * 32
        b8v = jnp.bitwise_and(lax.iota(jnp.int32, 16), 7)
        vconst = tbv32 + b8v

        def _pick(t, carry):
            ij = jnp.minimum(base16 + t, HW - 1)
            i = ij // H
            j = ij - i * H
            iw = (j * IN_HW) // H
            rr = i - i0
            row_idx = vconst + (rr * 64 + (iw // 128) * 8)
            col_idx = jnp.broadcast_to(iw % 128, (16,)).astype(jnp.int32)
            val_v[pl.ds(16 * t, 16)] = plsc.load_gather(
                buf, [row_idx, col_idx])
            return carry

        lax.fori_loop(0, WCHUNK // 16, _pick, 0)
        pltpu.sync_copy(val_v, out_hbm.at[pl.ds(wid * WCHUNK, WCHUNK)])

    pl.run_scoped(inner, pltpu.VMEM((256, 128), jnp.int32))


@jax.jit
def _sc_downsample(propT):
    fn = functools.partial(
        pl.kernel,
        mesh=plsc.VectorSubcoreMesh(core_axis_name="c", subcore_axis_name="s"),
        compiler_params=pltpu.CompilerParams(needs_layout_passes=False),
        out_type=jax.ShapeDtypeStruct((GOUT,), jnp.int32),
        scratch_types=[
            pltpu.VMEM((WCHUNK,), jnp.int32),
            pltpu.SemaphoreType.DMA,
        ],
    )(_sc_gather_body)
    return fn(propT)


HB = 5
NSTEP = H // HB
PB = HB * W * B


def _tc_body(feats_ref, prev_ref, lab_ref, out_ref, acc_anc, acc_con, cnt_ref):
    step = pl.program_id(0)

    @pl.when(step == 0)
    def _init():
        acc_anc[...] = jnp.zeros_like(acc_anc)
        acc_con[...] = jnp.zeros_like(acc_con)
        cnt_ref[...] = jnp.zeros_like(cnt_ref)

    lab = lab_ref[0]
    kiota = lax.broadcasted_iota(jnp.int32, (KPAD, PB), 0)
    oh = (lab == kiota).astype(jnp.float32)
    ones_c = jnp.ones((1, C), jnp.float32)
    ones_p = jnp.ones((PB, 1), jnp.float32)

    x = feats_ref[...].reshape(PB, C)
    xx = x * x
    sumsq = lax.dot_general(ones_c, xx, (((1,), (1,)), ((), ())),
                            preferred_element_type=jnp.float32)
    inv = 1.0 / jnp.maximum(jnp.sqrt(sumsq), 1e-12)
    acc_anc[...] += lax.dot_general(oh * inv, x, (((1,), (0,)), ((), ())),
                                    preferred_element_type=jnp.float32)

    xp = prev_ref[...].reshape(PB, C)
    xxp = xp * xp
    sumsqp = lax.dot_general(ones_c, xxp, (((1,), (1,)), ((), ())),
                             preferred_element_type=jnp.float32)
    invp = 1.0 / jnp.maximum(jnp.sqrt(sumsqp), 1e-12)
    acc_con[...] += lax.dot_general(oh * invp, xp, (((1,), (0,)), ((), ())),
                                    preferred_element_type=jnp.float32)

    cnt_ref[...] += jnp.dot(oh, ones_p,
                            preferred_element_type=jnp.float32)

    @pl.when(step == NSTEP - 1)
    def _finish():
        denom = jnp.maximum(cnt_ref[...], 1.0)
        kvalid = lax.broadcasted_iota(jnp.int32, (KPAD, 1), 0) < NUM
        anc = jnp.where(kvalid, acc_anc[...] / denom, 0.0)
        con = jnp.where(kvalid, acc_con[...] / denom, 0.0)
        contrast = jnp.concatenate([anc, con], axis=0)
        adc = lax.dot_general(anc, contrast, (((1,), (1,)), ((), ())),
                              preferred_element_type=jnp.float32) / TEMP

        ii = lax.broadcasted_iota(jnp.int32, (KPAD, 2 * KPAD), 0)
        jj = lax.broadcasted_iota(jnp.int32, (KPAD, 2 * KPAD), 1)
        jlab = jnp.where(jj < KPAD, jj, jj - KPAD)
        ivalid = ii < NUM
        jvalid = jlab < NUM
        vvalid = ivalid & jvalid
        r_mask = (vvalid & (ii == jlab)).astype(jnp.float32)
        eye = (vvalid & (jj < KPAD) & (ii == jj)).astype(jnp.float32)
        pos_mask = r_mask - eye
        neg_mask = jnp.where(vvalid, 1.0 - r_mask, 0.0)

        neg_contrast = jnp.sum(jnp.exp(adc) * neg_mask, axis=1, keepdims=True)
        logits_max = jnp.max(jnp.where(jvalid, adc, -1e30), axis=1,
                             keepdims=True)
        adc2 = adc - logits_max
        pos_contrast = (adc2 * pos_mask
                        - jnp.log(jnp.exp(adc2) + neg_contrast) * pos_mask)
        npos = jnp.sum(pos_mask, axis=1, keepdims=True)
        per = jnp.sum(pos_contrast, axis=1, keepdims=True)
        has = npos > 0
        loss_vec = jnp.where(has, -per / jnp.maximum(npos, 1.0), 0.0)
        num = jnp.sum(loss_vec, axis=0, keepdims=True)
        den = jnp.sum(has.astype(jnp.float32), axis=0, keepdims=True)
        out_ref[...] = num / jnp.maximum(den, 1.0)


@jax.jit
def _tc_main(featsT, featsT_prev, labels):
    return pl.pallas_call(
        _tc_body,
        grid=(NSTEP,),
        in_specs=[
            pl.BlockSpec((HB, W, B, C), lambda s: (s, 0, 0, 0)),
            pl.BlockSpec((HB, W, B, C), lambda s: (s, 0, 0, 0)),
            pl.BlockSpec((1, 1, PB), lambda s: (s, 0, 0)),
        ],
        out_specs=pl.BlockSpec((1, 1), lambda s: (0, 0)),
        out_shape=jax.ShapeDtypeStruct((1, 1), jnp.float32),
        scratch_shapes=[
            pltpu.VMEM((KPAD, C), jnp.float32),
            pltpu.VMEM((KPAD, C), jnp.float32),
            pltpu.VMEM((KPAD, 1), jnp.float32),
        ],
    )(featsT, featsT_prev, labels)


def kernel(pre_logits, pre_logits_prev, proposal):
    propT = jnp.transpose(proposal, (1, 0, 2))
    labels = _sc_downsample(propT)[:NPIX].reshape(NSTEP, 1, PB)
    featsT = jnp.transpose(pre_logits, (2, 3, 0, 1))
    featsT_prev = jnp.transpose(pre_logits_prev, (2, 3, 0, 1))
    return _tc_main(featsT, featsT_prev, labels)[0, 0]
